# Initial kernel scaffold; baseline (speedup 1.0000x reference)
import jax
import jax.numpy as jnp
from jax.experimental import pallas as pl


def kernel(x, speaker, duration, f0, rmse, position, max_y_len, emb_table, W, b):
    raise NotImplementedError("write your pallas kernel here")


# speedup vs baseline: 18.0738x; 18.0738x over previous
import jax
import jax.numpy as jnp
from jax import lax
from jax.experimental import pallas as pl
from jax.experimental.pallas import tpu as pltpu


def _body(x_ref, spk_ref, dur_ref, tails_ref, emb_ref, w1t_ref, w2t_ref,
          b_ref, out_ref):
    T, E = x_ref.shape[1], x_ref.shape[2]
    A = w1t_ref.shape[1]
    Y = out_ref.shape[1]
    S = emb_ref.shape[0]

    xb = x_ref[0]
    spk = spk_ref[0]
    oh = (spk == lax.broadcasted_iota(jnp.int32, (T, S), 1)).astype(jnp.float32)
    sp = jnp.dot(oh, emb_ref[...], preferred_element_type=jnp.float32)

    xl = (jnp.dot(xb, w1t_ref[...], preferred_element_type=jnp.float32)
          + jnp.dot(sp, w2t_ref[...], preferred_element_type=jnp.float32)
          + b_ref[...])

    durf = dur_ref[0].astype(jnp.float32)
    tri = (lax.broadcasted_iota(jnp.int32, (T, T), 0)
           <= lax.broadcasted_iota(jnp.int32, (T, T), 1)).astype(jnp.float32)
    cum = jnp.dot(durf, tri, preferred_element_type=jnp.float32)

    posy = lax.broadcasted_iota(jnp.int32, (Y, 1), 0).astype(jnp.float32)
    lo = (cum - durf <= posy).astype(jnp.float32)
    hi = (cum <= posy).astype(jnp.float32)
    P = lo - hi

    xe = jnp.dot(P, xl, preferred_element_type=jnp.float32)

    out_ref[0, :, :A] = xe
    out_ref[0, :, A:] = tails_ref[0]


def kernel(x, speaker, duration, f0, rmse, position, max_y_len, emb_table, W, b):
    B, T, E = x.shape
    Y = f0.shape[1]
    A, EK = W.shape
    S, K = emb_table.shape

    w1t = W[:, :E].T
    w2t = W[:, E:].T
    b_row = b.reshape(1, A)
    spk3 = speaker.reshape(B, T, 1)
    dur3 = duration.reshape(B, 1, T)
    tails = jnp.stack([f0, rmse, position], axis=-1)

    out = pl.pallas_call(
        _body,
        grid=(B,),
        in_specs=[
            pl.BlockSpec((1, T, E), lambda i: (i, 0, 0)),
            pl.BlockSpec((1, T, 1), lambda i: (i, 0, 0)),
            pl.BlockSpec((1, 1, T), lambda i: (i, 0, 0)),
            pl.BlockSpec((1, Y, 3), lambda i: (i, 0, 0)),
            pl.BlockSpec((S, K), lambda i: (0, 0)),
            pl.BlockSpec((E, A), lambda i: (0, 0)),
            pl.BlockSpec((K, A), lambda i: (0, 0)),
            pl.BlockSpec((1, A), lambda i: (0, 0)),
        ],
        out_specs=pl.BlockSpec((1, Y, A + 3), lambda i: (i, 0, 0)),
        out_shape=jax.ShapeDtypeStruct((B, Y, A + 3), jnp.float32),
        compiler_params=pltpu.CompilerParams(
            dimension_semantics=("arbitrary",)),
    )(x, spk3, dur3, tails, emb_table, w1t, w2t, b_row)
    return out
